# baseline (speedup 1.0000x reference)
import functools

import jax
import jax.numpy as jnp
from jax import lax
from jax.experimental import pallas as pl
from jax.experimental.pallas import tpu as pltpu
from jax.experimental.pallas import tpu_sc as plsc

_LANES = 16
_NW = 32
_CH = 16


def _sc_body(B, L, D, x_raw, pos_hbm, out_raw, xbuf, pbuf,
             sx0, sx1, sx2, st0, st1, st2, sp):
    x_hbm = x_raw.reshape(B * L, D)
    out_hbm = out_raw.reshape(B * L, D)
    cid = lax.axis_index("c")
    sid = lax.axis_index("s")
    wid = sid * 2 + cid
    pos_per_w = L // _NW
    n_chunks = pos_per_w // _CH
    pos_lo = wid * pos_per_w

    sem_x = [sx0, sx1, sx2]
    sem_st = [st0, st1, st2]
    steps = [(c, b) for c in range(n_chunks) for b in range(B)]
    n_steps = len(steps)

    def x_row(c, b):
        return b * L + pos_lo + c * _CH

    descs = {}

    def start_load_x(i):
        c, b = steps[i]
        descs["x", i] = pltpu.async_copy(
            x_hbm.at[pl.ds(x_row(c, b), _CH), :], xbuf.at[i % 3],
            sem_x[i % 3])

    def start_load_p(c):
        descs["p", c] = pltpu.async_copy(
            pos_hbm.at[pl.ds(pos_lo + c * _CH, _CH), :], pbuf.at[c % 2], sp)

    def start_store(i):
        c, b = steps[i]
        descs["st", i] = pltpu.async_copy(
            xbuf.at[i % 3], out_hbm.at[pl.ds(x_row(c, b), _CH), :],
            sem_st[i % 3])

    start_load_p(0)
    start_load_x(0)
    for i, (c, b) in enumerate(steps):
        slot = i % 3
        if b == 0:
            descs["p", c].wait()
            if c + 1 < n_chunks:
                start_load_p(c + 1)
        if i + 1 < n_steps:
            if i + 1 >= 3:
                descs["st", i - 2].wait()
            start_load_x(i + 1)
        descs["x", i].wait()

        n_sl = D // _LANES

        def add_slice(t, slot=slot, pslot=c % 2):
            r = t // n_sl
            sl = pl.ds(lax.rem(t, n_sl) * _LANES, _LANES)
            plsc.addupdate(xbuf.at[slot, r, sl], pbuf[pslot, r, sl])

        plsc.parallel_loop(0, _CH * n_sl, 1, unroll=16)(add_slice)
        start_store(i)
    for i in range(max(0, n_steps - 3), n_steps):
        descs["st", i].wait()


def kernel(x, pos_table):
    B, L, D = x.shape

    mesh = plsc.VectorSubcoreMesh(core_axis_name="c", subcore_axis_name="s")
    sc = pl.kernel(
        functools.partial(_sc_body, B, L, D),
        out_type=jax.ShapeDtypeStruct((B, L, D), jnp.float32),
        mesh=mesh,
        scratch_types=[
            pltpu.VMEM((3, _CH, D), jnp.float32),
            pltpu.VMEM((2, _CH, D), jnp.float32),
            pltpu.SemaphoreType.DMA,
            pltpu.SemaphoreType.DMA,
            pltpu.SemaphoreType.DMA,
            pltpu.SemaphoreType.DMA,
            pltpu.SemaphoreType.DMA,
            pltpu.SemaphoreType.DMA,
            pltpu.SemaphoreType.DMA,
        ],
    )
    return sc(x, pos_table)
